# baseline (speedup 1.0000x reference)
import jax
import jax.numpy as jnp
import numpy as np
from jax.experimental import pallas as pl
from jax.experimental.pallas import tpu as pltpu

_SCALE = 2.0 ** -1.5
_DHALF = 64


def _haar_ll_kernel(x0_ref, x1_ref, p_ref, o_ref):
    p = p_ref[...]
    ev = pl.ds(0, 64, 2)
    od = pl.ds(1, 64, 2)
    for half, x_ref in enumerate((x0_ref, x1_ref)):
        for k in range(_DHALF // 2):
            s = (
                x_ref[0, 2 * k, ev, :]
                + x_ref[0, 2 * k, od, :]
                + x_ref[0, 2 * k + 1, ev, :]
                + x_ref[0, 2 * k + 1, od, :]
            )
            o_ref[0, half * (_DHALF // 2) + k] = jnp.dot(
                s, p, preferred_element_type=jnp.float32
            )


@jax.jit
def kernel(x):
    B, C, D, H, W = x.shape
    n = B * C
    xr = x.reshape(n, D, H, W)
    rows = np.arange(W) // 2
    p = (rows[:, None] == np.arange(W // 2)[None, :]).astype(np.float32) * _SCALE
    p = jnp.asarray(p)

    out = pl.pallas_call(
        _haar_ll_kernel,
        grid=(n,),
        in_specs=[
            pl.BlockSpec((1, _DHALF, H, W), lambda i: (i, 0, 0, 0)),
            pl.BlockSpec((1, _DHALF, H, W), lambda i: (i, 1, 0, 0)),
            pl.BlockSpec((W, W // 2), lambda i: (0, 0)),
        ],
        out_specs=pl.BlockSpec((1, D // 2, H // 2, W // 2), lambda i: (i, 0, 0, 0)),
        out_shape=jax.ShapeDtypeStruct((n, D // 2, H // 2, W // 2), x.dtype),
        compiler_params=pltpu.CompilerParams(
            dimension_semantics=("parallel",),
        ),
    )(xr, xr, p)
    return out.reshape(B, C, D // 2, H // 2, W // 2)
